# baseline (speedup 1.0000x reference)
import functools

import jax
import jax.numpy as jnp
from jax import lax
from jax.experimental import pallas as pl
from jax.experimental.pallas import tpu as pltpu
from jax.experimental.pallas import tpu_sc as plsc

GRAPH_BIAS_SCALE = 0.8
CONTEXT_GATE_STRENGTH = 2.0
COMMITMENT_COST = 0.25

N = 8192
D = 256
BN = 256


def _vq_block(z, w, b, cb2, cn):
    zn = jnp.sum(z * z, axis=1, keepdims=True)
    zc2 = jax.lax.dot_general(z, cb2, (((1,), (1,)), ((), ())),
                              preferred_element_type=jnp.float32)
    logits = jnp.dot(z, w, preferred_element_type=jnp.float32) + b
    e = jnp.exp(logits)
    g = (CONTEXT_GATE_STRENGTH / jnp.sum(e, axis=1, keepdims=True)) * e
    d = (zn + cn - zc2) - g
    idx = jnp.argmin(d, axis=1).astype(jnp.int32)
    d_min = jnp.min(d, axis=1)
    g_idx = jnp.max(jnp.where(d == d_min[:, None], g, -1.0), axis=1)
    return idx, jnp.sum(d_min + g_idx)


def _body(zf_ref, zs_ref, cbsyn_ref, cbsem_ref, wsyn_ref, bsyn_ref,
          wsem_ref, bsem_ref,
          idxsyn_ref, idxsem_ref, loss_ref,
          cb2syn_ref, cnsyn_ref, cb2sem_ref, cnsem_ref):
    step = pl.program_id(0)
    nsteps = pl.num_programs(0)

    @pl.when(step == 0)
    def _():
        loss_ref[0, 0] = 0.0
        cb_s = cbsyn_ref[...]
        cb2syn_ref[...] = cb_s + cb_s
        cnsyn_ref[...] = jnp.sum(cb_s * cb_s, axis=1)[None, :]
        cb_m = cbsem_ref[...]
        cb2sem_ref[...] = cb_m + cb_m
        cnsem_ref[...] = jnp.sum(cb_m * cb_m, axis=1)[None, :]

    idx_s, ss_s = _vq_block(zf_ref[...], wsyn_ref[...], bsyn_ref[...],
                            cb2syn_ref[...], cnsyn_ref[...])
    idxsyn_ref[...] = idx_s
    idx_m, ss_m = _vq_block(zs_ref[...], wsem_ref[...], bsem_ref[...],
                            cb2sem_ref[...], cnsem_ref[...])
    idxsem_ref[...] = idx_m

    loss_ref[0, 0] += ss_s + ss_m

    @pl.when(step == nsteps - 1)
    def _():
        loss_ref[0, 0] = loss_ref[0, 0] * ((1.0 + COMMITMENT_COST) / (N * D))


def _tc_call(z_fast, z_slow, cb_syn, cb_sem, Wg_syn, bg_syn, Wg_sem, bg_sem):
    n_syn = cb_syn.shape[0]
    n_sem = cb_sem.shape[0]
    grid = (N // BN,)

    row_spec = pl.BlockSpec((BN, D), lambda i: (i, 0))
    full = lambda shape: pl.BlockSpec(shape, lambda i: (0,) * len(shape))

    out_shapes = (
        jax.ShapeDtypeStruct((N,), jnp.int32),
        jax.ShapeDtypeStruct((N,), jnp.int32),
        jax.ShapeDtypeStruct((1, 1), jnp.float32),
    )
    out_specs = (
        pl.BlockSpec((BN,), lambda i: (i,)),
        pl.BlockSpec((BN,), lambda i: (i,)),
        pl.BlockSpec((1, 1), lambda i: (0, 0), memory_space=pltpu.SMEM),
    )
    in_specs = [
        row_spec,
        row_spec,
        full((n_syn, D)),
        full((n_sem, D)),
        full((D, n_syn)),
        full((1, n_syn)),
        full((D, n_sem)),
        full((1, n_sem)),
    ]
    scratch_shapes = [
        pltpu.VMEM((n_syn, D), jnp.float32),
        pltpu.VMEM((1, n_syn), jnp.float32),
        pltpu.VMEM((n_sem, D), jnp.float32),
        pltpu.VMEM((1, n_sem), jnp.float32),
    ]

    return pl.pallas_call(
        _body,
        grid=grid,
        in_specs=in_specs,
        out_specs=out_specs,
        out_shape=out_shapes,
        scratch_shapes=scratch_shapes,
    )(z_fast, z_slow, cb_syn, cb_sem, Wg_syn, bg_syn.reshape(1, n_syn),
      Wg_sem, bg_sem.reshape(1, n_sem))


def _sc_gather(cb_syn, cb_sem, idx_syn, idx_sem):
    info = plsc.get_sparse_core_info()
    nw = info.num_cores * info.num_subcores
    b_per_w = N // nw
    mesh = plsc.VectorSubcoreMesh(core_axis_name="c", subcore_axis_name="s")

    @functools.partial(
        pl.kernel,
        mesh=mesh,
        out_type=(jax.ShapeDtypeStruct((N, D), jnp.float32),
                  jax.ShapeDtypeStruct((N, D), jnp.float32)),
        scratch_types=[
            pltpu.VMEM((b_per_w,), jnp.int32),
            pltpu.VMEM((b_per_w, D), jnp.float32),
            pltpu.SemaphoreType.DMA,
        ],
    )
    def _k(cbsyn_hbm, cbsem_hbm, idxsyn_hbm, idxsem_hbm,
           outsyn_hbm, outsem_hbm, idx_v, rows_v, sem):
        wid = lax.axis_index("s") * info.num_cores + lax.axis_index("c")
        base = wid * b_per_w
        pltpu.sync_copy(idxsyn_hbm.at[pl.ds(base, b_per_w)], idx_v)
        pltpu.async_copy(cbsyn_hbm.at[idx_v], rows_v, sem).wait()
        pltpu.sync_copy(rows_v, outsyn_hbm.at[pl.ds(base, b_per_w)])
        pltpu.sync_copy(idxsem_hbm.at[pl.ds(base, b_per_w)], idx_v)
        pltpu.async_copy(cbsem_hbm.at[idx_v], rows_v, sem).wait()
        pltpu.sync_copy(rows_v, outsem_hbm.at[pl.ds(base, b_per_w)])

    return _k(cb_syn, cb_sem, idx_syn, idx_sem)


def kernel(z_fast, z_slow, cb_syn, cb_sem, Wg_syn, bg_syn, Wg_sem, bg_sem):
    idx_syn, idx_sem, loss = _tc_call(z_fast, z_slow, cb_syn, cb_sem,
                                      Wg_syn, bg_syn, Wg_sem, bg_sem)
    zq_syn, zq_sem = _sc_gather(cb_syn, cb_sem, idx_syn, idx_sem)
    half = D // 2
    zqc_syn = jax.lax.complex(zq_syn[:, :half], zq_syn[:, half:])
    zqc_sem = jax.lax.complex(zq_sem[:, :half], zq_sem[:, half:])
    return (zqc_syn, zqc_sem, loss[0, 0], (idx_syn, idx_sem))


# speedup vs baseline: 1.0058x; 1.0058x over previous
import functools

import jax
import jax.numpy as jnp
from jax import lax
from jax.experimental import pallas as pl
from jax.experimental.pallas import tpu as pltpu
from jax.experimental.pallas import tpu_sc as plsc

GRAPH_BIAS_SCALE = 0.8
CONTEXT_GATE_STRENGTH = 2.0
COMMITMENT_COST = 0.25

N = 8192
D = 256
BN = 256


def _vq_block(z, w, b, cb2, cn):
    zn = jnp.sum(z * z, axis=1, keepdims=True)
    zc2 = jax.lax.dot_general(z, cb2, (((1,), (1,)), ((), ())),
                              preferred_element_type=jnp.float32)
    logits = jnp.dot(z, w, preferred_element_type=jnp.float32) + b
    e = jnp.exp(logits)
    g = (CONTEXT_GATE_STRENGTH / jnp.sum(e, axis=1, keepdims=True)) * e
    d = (zn + cn - zc2) - g
    idx = jnp.argmin(d, axis=1).astype(jnp.int32)
    d_min = jnp.min(d, axis=1)
    g_idx = jnp.max(jnp.where(d == d_min[:, None], g, -1.0), axis=1)
    return idx, jnp.sum(d_min + g_idx)


def _body(zf_ref, zs_ref, cbsyn_ref, cbsem_ref, wsyn_ref, bsyn_ref,
          wsem_ref, bsem_ref,
          idxsyn_ref, idxsem_ref, loss_ref,
          cb2syn_ref, cnsyn_ref, cb2sem_ref, cnsem_ref):
    step = pl.program_id(0)
    nsteps = pl.num_programs(0)

    @pl.when(step == 0)
    def _():
        loss_ref[0, 0] = 0.0
        cb_s = cbsyn_ref[...]
        cb2syn_ref[...] = cb_s + cb_s
        cnsyn_ref[...] = jnp.sum(cb_s * cb_s, axis=1)[None, :]
        cb_m = cbsem_ref[...]
        cb2sem_ref[...] = cb_m + cb_m
        cnsem_ref[...] = jnp.sum(cb_m * cb_m, axis=1)[None, :]

    idx_s, ss_s = _vq_block(zf_ref[...], wsyn_ref[...], bsyn_ref[...],
                            cb2syn_ref[...], cnsyn_ref[...])
    idxsyn_ref[...] = idx_s
    idx_m, ss_m = _vq_block(zs_ref[...], wsem_ref[...], bsem_ref[...],
                            cb2sem_ref[...], cnsem_ref[...])
    idxsem_ref[...] = idx_m

    loss_ref[0, 0] += ss_s + ss_m

    @pl.when(step == nsteps - 1)
    def _():
        loss_ref[0, 0] = loss_ref[0, 0] * ((1.0 + COMMITMENT_COST) / (N * D))


def _tc_call(z_fast, z_slow, cb_syn, cb_sem, Wg_syn, bg_syn, Wg_sem, bg_sem):
    n_syn = cb_syn.shape[0]
    n_sem = cb_sem.shape[0]
    grid = (N // BN,)

    row_spec = pl.BlockSpec((BN, D), lambda i: (i, 0))
    full = lambda shape: pl.BlockSpec(shape, lambda i: (0,) * len(shape))

    out_shapes = (
        jax.ShapeDtypeStruct((N,), jnp.int32),
        jax.ShapeDtypeStruct((N,), jnp.int32),
        jax.ShapeDtypeStruct((1, 1), jnp.float32),
    )
    out_specs = (
        pl.BlockSpec((BN,), lambda i: (i,)),
        pl.BlockSpec((BN,), lambda i: (i,)),
        pl.BlockSpec((1, 1), lambda i: (0, 0), memory_space=pltpu.SMEM),
    )
    in_specs = [
        row_spec,
        row_spec,
        full((n_syn, D)),
        full((n_sem, D)),
        full((D, n_syn)),
        full((1, n_syn)),
        full((D, n_sem)),
        full((1, n_sem)),
    ]
    scratch_shapes = [
        pltpu.VMEM((n_syn, D), jnp.float32),
        pltpu.VMEM((1, n_syn), jnp.float32),
        pltpu.VMEM((n_sem, D), jnp.float32),
        pltpu.VMEM((1, n_sem), jnp.float32),
    ]

    return pl.pallas_call(
        _body,
        grid=grid,
        in_specs=in_specs,
        out_specs=out_specs,
        out_shape=out_shapes,
        scratch_shapes=scratch_shapes,
    )(z_fast, z_slow, cb_syn, cb_sem, Wg_syn, bg_syn.reshape(1, n_syn),
      Wg_sem, bg_sem.reshape(1, n_sem))


def _sc_gather(cb_syn, cb_sem, idx_syn, idx_sem):
    info = plsc.get_sparse_core_info()
    nw = info.num_cores * info.num_subcores
    b_per_w = N // nw
    mesh = plsc.VectorSubcoreMesh(core_axis_name="c", subcore_axis_name="s")

    @functools.partial(
        pl.kernel,
        mesh=mesh,
        out_type=(jax.ShapeDtypeStruct((N, D), jnp.float32),
                  jax.ShapeDtypeStruct((N, D), jnp.float32)),
        scratch_types=[
            pltpu.VMEM((b_per_w,), jnp.int32),
            pltpu.VMEM((b_per_w, D), jnp.float32),
            pltpu.SemaphoreType.DMA,
        ],
    )
    def _k(cbsyn_hbm, cbsem_hbm, idxsyn_hbm, idxsem_hbm,
           outsyn_hbm, outsem_hbm, idx_v, rows_v, sem):
        wid = lax.axis_index("s") * info.num_cores + lax.axis_index("c")
        base = wid * b_per_w
        pltpu.sync_copy(idxsyn_hbm.at[pl.ds(base, b_per_w)], idx_v)
        pltpu.async_copy(cbsyn_hbm.at[idx_v], rows_v, sem).wait()
        pltpu.sync_copy(rows_v, outsyn_hbm.at[pl.ds(base, b_per_w)])
        pltpu.sync_copy(idxsem_hbm.at[pl.ds(base, b_per_w)], idx_v)
        pltpu.async_copy(cbsem_hbm.at[idx_v], rows_v, sem).wait()
        pltpu.sync_copy(rows_v, outsem_hbm.at[pl.ds(base, b_per_w)])

    return _k(cb_syn, cb_sem, idx_syn, idx_sem)


def kernel(z_fast, z_slow, cb_syn, cb_sem, Wg_syn, bg_syn, Wg_sem, bg_sem):
    idx_syn, idx_sem, loss = _tc_call(z_fast, z_slow, cb_syn, cb_sem,
                                      Wg_syn, bg_syn, Wg_sem, bg_sem)
    zq_syn, zq_sem = _sc_gather(cb_syn, cb_sem, idx_syn, idx_sem)
    half = D // 2
    j1 = jnp.complex64(1j)
    zqc_syn = (zq_syn[:, :half].astype(jnp.complex64)
               + zq_syn[:, half:].astype(jnp.complex64) * j1)
    zqc_sem = (zq_sem[:, :half].astype(jnp.complex64)
               + zq_sem[:, half:].astype(jnp.complex64) * j1)
    return (zqc_syn, zqc_sem, loss[0, 0], (idx_syn, idx_sem))


# speedup vs baseline: 1.0672x; 1.0610x over previous
import functools

import jax
import jax.numpy as jnp
from jax import lax
from jax.experimental import pallas as pl
from jax.experimental.pallas import tpu as pltpu
from jax.experimental.pallas import tpu_sc as plsc

GRAPH_BIAS_SCALE = 0.8
CONTEXT_GATE_STRENGTH = 2.0
COMMITMENT_COST = 0.25

N = 8192
D = 256
BN = 1024


def _vq_block(z, w, b, cb2, cn):
    zn = jnp.sum(z * z, axis=1, keepdims=True)
    zc2 = jax.lax.dot_general(z, cb2, (((1,), (1,)), ((), ())),
                              preferred_element_type=jnp.float32)
    logits = jnp.dot(z, w, preferred_element_type=jnp.float32) + b
    e = jnp.exp(logits)
    g = (CONTEXT_GATE_STRENGTH / jnp.sum(e, axis=1, keepdims=True)) * e
    d = (zn + cn - zc2) - g
    k = d.shape[1]
    d_min = jnp.min(d, axis=1)
    eq = d == d_min[:, None]
    iota = jax.lax.broadcasted_iota(jnp.int32, d.shape, 1)
    idx = jnp.min(jnp.where(eq, iota, k), axis=1).astype(jnp.int32)
    g_idx = jnp.max(jnp.where(eq, g, -1.0), axis=1)
    return idx, jnp.sum(d_min + g_idx)


def _body(zf_ref, zs_ref, cbsyn_ref, cbsem_ref, wsyn_ref, bsyn_ref,
          wsem_ref, bsem_ref,
          idxsyn_ref, idxsem_ref, loss_ref,
          cb2syn_ref, cnsyn_ref, cb2sem_ref, cnsem_ref):
    step = pl.program_id(0)
    nsteps = pl.num_programs(0)

    @pl.when(step == 0)
    def _():
        loss_ref[0, 0] = 0.0
        cb_s = cbsyn_ref[...]
        cb2syn_ref[...] = cb_s + cb_s
        cnsyn_ref[...] = jnp.sum(cb_s * cb_s, axis=1)[None, :]
        cb_m = cbsem_ref[...]
        cb2sem_ref[...] = cb_m + cb_m
        cnsem_ref[...] = jnp.sum(cb_m * cb_m, axis=1)[None, :]

    idx_s, ss_s = _vq_block(zf_ref[...], wsyn_ref[...], bsyn_ref[...],
                            cb2syn_ref[...], cnsyn_ref[...])
    idxsyn_ref[...] = idx_s
    idx_m, ss_m = _vq_block(zs_ref[...], wsem_ref[...], bsem_ref[...],
                            cb2sem_ref[...], cnsem_ref[...])
    idxsem_ref[...] = idx_m

    loss_ref[0, 0] += ss_s + ss_m

    @pl.when(step == nsteps - 1)
    def _():
        loss_ref[0, 0] = loss_ref[0, 0] * ((1.0 + COMMITMENT_COST) / (N * D))


def _tc_call(z_fast, z_slow, cb_syn, cb_sem, Wg_syn, bg_syn, Wg_sem, bg_sem):
    n_syn = cb_syn.shape[0]
    n_sem = cb_sem.shape[0]
    grid = (N // BN,)

    row_spec = pl.BlockSpec((BN, D), lambda i: (i, 0))
    full = lambda shape: pl.BlockSpec(shape, lambda i: (0,) * len(shape))

    out_shapes = (
        jax.ShapeDtypeStruct((N,), jnp.int32),
        jax.ShapeDtypeStruct((N,), jnp.int32),
        jax.ShapeDtypeStruct((1, 1), jnp.float32),
    )
    out_specs = (
        pl.BlockSpec((BN,), lambda i: (i,)),
        pl.BlockSpec((BN,), lambda i: (i,)),
        pl.BlockSpec((1, 1), lambda i: (0, 0), memory_space=pltpu.SMEM),
    )
    in_specs = [
        row_spec,
        row_spec,
        full((n_syn, D)),
        full((n_sem, D)),
        full((D, n_syn)),
        full((1, n_syn)),
        full((D, n_sem)),
        full((1, n_sem)),
    ]
    scratch_shapes = [
        pltpu.VMEM((n_syn, D), jnp.float32),
        pltpu.VMEM((1, n_syn), jnp.float32),
        pltpu.VMEM((n_sem, D), jnp.float32),
        pltpu.VMEM((1, n_sem), jnp.float32),
    ]

    return pl.pallas_call(
        _body,
        grid=grid,
        in_specs=in_specs,
        out_specs=out_specs,
        out_shape=out_shapes,
        scratch_shapes=scratch_shapes,
    )(z_fast, z_slow, cb_syn, cb_sem, Wg_syn, bg_syn.reshape(1, n_syn),
      Wg_sem, bg_sem.reshape(1, n_sem))


def _sc_gather(cb_syn, cb_sem, idx_syn, idx_sem):
    info = plsc.get_sparse_core_info()
    nw = info.num_cores * info.num_subcores
    b_per_w = N // nw
    mesh = plsc.VectorSubcoreMesh(core_axis_name="c", subcore_axis_name="s")

    @functools.partial(
        pl.kernel,
        mesh=mesh,
        out_type=(jax.ShapeDtypeStruct((N, D), jnp.float32),
                  jax.ShapeDtypeStruct((N, D), jnp.float32)),
        scratch_types=[
            pltpu.VMEM((b_per_w,), jnp.int32),
            pltpu.VMEM((b_per_w, D), jnp.float32),
            pltpu.SemaphoreType.DMA,
        ],
    )
    def _k(cbsyn_hbm, cbsem_hbm, idxsyn_hbm, idxsem_hbm,
           outsyn_hbm, outsem_hbm, idx_v, rows_v, sem):
        wid = lax.axis_index("s") * info.num_cores + lax.axis_index("c")
        base = wid * b_per_w
        pltpu.sync_copy(idxsyn_hbm.at[pl.ds(base, b_per_w)], idx_v)
        pltpu.async_copy(cbsyn_hbm.at[idx_v], rows_v, sem).wait()
        pltpu.sync_copy(rows_v, outsyn_hbm.at[pl.ds(base, b_per_w)])
        pltpu.sync_copy(idxsem_hbm.at[pl.ds(base, b_per_w)], idx_v)
        pltpu.async_copy(cbsem_hbm.at[idx_v], rows_v, sem).wait()
        pltpu.sync_copy(rows_v, outsem_hbm.at[pl.ds(base, b_per_w)])

    return _k(cb_syn, cb_sem, idx_syn, idx_sem)


def kernel(z_fast, z_slow, cb_syn, cb_sem, Wg_syn, bg_syn, Wg_sem, bg_sem):
    idx_syn, idx_sem, loss = _tc_call(z_fast, z_slow, cb_syn, cb_sem,
                                      Wg_syn, bg_syn, Wg_sem, bg_sem)
    zq_syn, zq_sem = _sc_gather(cb_syn, cb_sem, idx_syn, idx_sem)
    half = D // 2
    j1 = jnp.complex64(1j)
    zqc_syn = (zq_syn[:, :half].astype(jnp.complex64)
               + zq_syn[:, half:].astype(jnp.complex64) * j1)
    zqc_sem = (zq_sem[:, :half].astype(jnp.complex64)
               + zq_sem[:, half:].astype(jnp.complex64) * j1)
    return (zqc_syn, zqc_sem, loss[0, 0], (idx_syn, idx_sem))


# speedup vs baseline: 1.2843x; 1.2035x over previous
import functools

import jax
import jax.numpy as jnp
from jax import lax
from jax.experimental import pallas as pl
from jax.experimental.pallas import tpu as pltpu
from jax.experimental.pallas import tpu_sc as plsc

GRAPH_BIAS_SCALE = 0.8
CONTEXT_GATE_STRENGTH = 2.0
COMMITMENT_COST = 0.25

N = 8192
D = 256
BN = 1024


def _body(z_ref, cb_ref, w_ref, b_ref, idx_ref, loss_ref, cb2_ref, cn_ref):
    step = pl.program_id(0)

    @pl.when(step == 0)
    def _():
        loss_ref[0, 0] = 0.0
        cb = cb_ref[...]
        cb2_ref[...] = cb + cb
        cn_ref[...] = jnp.sum(cb * cb, axis=1)[None, :]

    z = z_ref[...]
    zn = jnp.sum(z * z, axis=1, keepdims=True)
    zc2 = jax.lax.dot_general(z, cb2_ref[...], (((1,), (1,)), ((), ())),
                              preferred_element_type=jnp.float32)
    logits = jnp.dot(z, w_ref[...], preferred_element_type=jnp.float32) + b_ref[...]
    e = jnp.exp(logits)
    g = (CONTEXT_GATE_STRENGTH / jnp.sum(e, axis=1, keepdims=True)) * e
    d = (zn + cn_ref[...] - zc2) - g
    k = d.shape[1]
    d_min = jnp.min(d, axis=1)
    eq = d == d_min[:, None]
    iota = jax.lax.broadcasted_iota(jnp.int32, d.shape, 1)
    idx_ref[...] = jnp.min(jnp.where(eq, iota, k), axis=1).astype(jnp.int32)
    g_idx = jnp.max(jnp.where(eq, g, -1.0), axis=1)
    loss_ref[0, 0] += jnp.sum(d_min + g_idx)


def _tc_branch(z, cb, w, b):
    k = cb.shape[0]
    grid = (N // BN,)
    full = lambda shape: pl.BlockSpec(shape, lambda i: (0,) * len(shape))

    idx, losssum = pl.pallas_call(
        _body,
        grid=grid,
        in_specs=[
            pl.BlockSpec((BN, D), lambda i: (i, 0)),
            full((k, D)),
            full((D, k)),
            full((1, k)),
        ],
        out_specs=(
            pl.BlockSpec((BN,), lambda i: (i,)),
            pl.BlockSpec((1, 1), lambda i: (0, 0), memory_space=pltpu.SMEM),
        ),
        out_shape=(
            jax.ShapeDtypeStruct((N,), jnp.int32),
            jax.ShapeDtypeStruct((1, 1), jnp.float32),
        ),
        scratch_shapes=[
            pltpu.VMEM((k, D), jnp.float32),
            pltpu.VMEM((1, k), jnp.float32),
        ],
    )(z, cb, w, b.reshape(1, k))
    return idx, losssum


def _sc_gather(cb, idx):
    info = plsc.get_sparse_core_info()
    nw = info.num_cores * info.num_subcores
    b_per_w = N // nw
    mesh = plsc.VectorSubcoreMesh(core_axis_name="c", subcore_axis_name="s")

    @functools.partial(
        pl.kernel,
        mesh=mesh,
        out_type=jax.ShapeDtypeStruct((N, D), jnp.float32),
        scratch_types=[
            pltpu.VMEM((b_per_w,), jnp.int32),
            pltpu.VMEM((b_per_w, D), jnp.float32),
            pltpu.SemaphoreType.DMA,
        ],
    )
    def _k(cb_hbm, idx_hbm, out_hbm, idx_v, rows_v, sem):
        wid = lax.axis_index("s") * info.num_cores + lax.axis_index("c")
        base = wid * b_per_w
        pltpu.sync_copy(idx_hbm.at[pl.ds(base, b_per_w)], idx_v)
        pltpu.async_copy(cb_hbm.at[idx_v], rows_v, sem).wait()
        pltpu.sync_copy(rows_v, out_hbm.at[pl.ds(base, b_per_w)])

    return _k(cb, idx)


def kernel(z_fast, z_slow, cb_syn, cb_sem, Wg_syn, bg_syn, Wg_sem, bg_sem):
    idx_syn, ls = _tc_branch(z_fast, cb_syn, Wg_syn, bg_syn)
    idx_sem, lm = _tc_branch(z_slow, cb_sem, Wg_sem, bg_sem)
    zq_syn = _sc_gather(cb_syn, idx_syn)
    zq_sem = _sc_gather(cb_sem, idx_sem)
    loss = (ls[0, 0] + lm[0, 0]) * ((1.0 + COMMITMENT_COST) / (N * D))
    half = D // 2
    zqc_syn = jax.lax.complex(zq_syn[:, :half], zq_syn[:, half:])
    zqc_sem = jax.lax.complex(zq_sem[:, :half], zq_sem[:, half:])
    return (zqc_syn, zqc_sem, loss, (idx_syn, idx_sem))
